# baseline (speedup 1.0000x reference)
import jax
import jax.numpy as jnp
from jax.experimental import pallas as pl
from jax.experimental.pallas import tpu as pltpu

_LN_EPS = 1e-5
_BM = 128


def _embed_kernel(x_ref, emb_ref, scale_ref, bias_ref, out_ref):
    xb = x_ref[...].astype(jnp.bfloat16)
    h = jnp.dot(xb, emb_ref[...], preferred_element_type=jnp.float32)
    h = jax.nn.gelu(h)
    mean = jnp.mean(h, axis=-1, keepdims=True)
    var = jnp.mean((h - mean) ** 2, axis=-1, keepdims=True)
    out_ref[...] = (h - mean) * jax.lax.rsqrt(var + _LN_EPS) * scale_ref[...] + bias_ref[...]


def kernel(x, emb, ln_scale, ln_bias):
    B, G = x.shape
    H = emb.shape[1]
    emb_bf = emb.astype(jnp.bfloat16)
    scale2 = ln_scale.reshape(1, H)
    bias2 = ln_bias.reshape(1, H)
    out = pl.pallas_call(
        _embed_kernel,
        grid=(B // _BM,),
        in_specs=[
            pl.BlockSpec((_BM, G), lambda i: (i, 0)),
            pl.BlockSpec((G, H), lambda i: (0, 0)),
            pl.BlockSpec((1, H), lambda i: (0, 0)),
            pl.BlockSpec((1, H), lambda i: (0, 0)),
        ],
        out_specs=pl.BlockSpec((_BM, H), lambda i: (i, 0)),
        out_shape=jax.ShapeDtypeStruct((B, H), jnp.float32),
        compiler_params=pltpu.CompilerParams(dimension_semantics=("parallel",)),
    )(x, emb_bf, scale2, bias2)
    gene_idx = jnp.arange(G, dtype=jnp.int32)
    return (out, gene_idx)
